# baseline (speedup 1.0000x reference)
import functools

import jax
import jax.numpy as jnp
from jax import lax
from jax.experimental import pallas as pl
from jax.experimental.pallas import tpu as pltpu
from jax.experimental.pallas import tpu_sc as plsc

_EPS = 1e-8
_NEG = -1e30


def _transpose_body(flag_ref, x_ref, o_ref, dest_ref, n0_ref, *, HW):
    b = pl.program_id(0)
    i = pl.program_id(1)

    @pl.when((b == 0) & (i == 0))
    def _ranks():
        z = jnp.where(flag_ref[...] < 0.5, 1.0, 0.0)
        cz = z
        sh = 1
        while sh < HW:
            shifted = jnp.concatenate(
                [jnp.zeros((1, sh), jnp.float32), cz[:, :HW - sh]], axis=1)
            cz = cz + shifted
            sh *= 2
        rank0 = cz - z
        total0 = jnp.sum(z)
        pos = lax.broadcasted_iota(jnp.int32, (1, HW), 1).astype(jnp.float32)
        dest = jnp.where(z > 0.5, rank0, total0 + (pos - rank0))
        dest_ref[...] = dest.astype(jnp.int32)
        n0_ref[...] = jnp.full((1, 8), total0, jnp.float32).astype(jnp.int32)

    o_ref[0] = jnp.swapaxes(x_ref[0], 0, 1)


def _latter_rowmajor(inp_chw, qflag, *, bqt=512):
    B, C, HW = inp_chw.shape
    c = C // 2
    return pl.pallas_call(
        functools.partial(_transpose_body, HW=HW),
        grid=(B, HW // bqt),
        in_specs=[
            pl.BlockSpec((1, HW), lambda b, i: (0, 0)),
            pl.BlockSpec((1, c, bqt), lambda b, i: (b, 1, i)),
        ],
        out_specs=[
            pl.BlockSpec((1, bqt, c), lambda b, i: (b, i, 0)),
            pl.BlockSpec((1, HW), lambda b, i: (0, 0)),
            pl.BlockSpec((1, 8), lambda b, i: (0, 0)),
        ],
        out_shape=[
            jax.ShapeDtypeStruct((B, HW, c), jnp.float32),
            jax.ShapeDtypeStruct((1, HW), jnp.int32),
            jax.ShapeDtypeStruct((1, 8), jnp.int32),
        ],
        compiler_params=pltpu.CompilerParams(
            dimension_semantics=("arbitrary", "arbitrary"),
        ),
    )(qflag, inp_chw)


def _sc_compact_scatter(dest, lat_flat, *, B, HW, c):
    info = plsc.get_sparse_core_info()
    NC, NS = info.num_cores, info.num_subcores
    NW = NC * NS
    rows_w = HW // NW

    mesh = plsc.VectorSubcoreMesh(core_axis_name="c", subcore_axis_name="s")

    @functools.partial(
        pl.kernel, mesh=mesh,
        out_type=jax.ShapeDtypeStruct((B * HW, c), jnp.float32),
        scratch_types=[
            pltpu.VMEM((rows_w,), jnp.int32),
            pltpu.VMEM((rows_w,), jnp.int32),
            pltpu.VMEM((rows_w, c), jnp.float32),
            pltpu.SemaphoreType.DMA,
        ],
    )
    def _k(dest_hbm, lat_hbm, out_hbm, dest_v, dest2_v, rows_v, sem):
        wid = lax.axis_index("s") * NC + lax.axis_index("c")
        base = pl.multiple_of(wid * rows_w, rows_w)
        pltpu.sync_copy(dest_hbm.at[pl.ds(base, rows_w)], dest_v)
        for b in range(B):
            pltpu.sync_copy(lat_hbm.at[pl.ds(b * HW + base, rows_w)], rows_v)
            if b == 0:
                pltpu.async_copy(rows_v, out_hbm.at[dest_v], sem).wait()
            else:
                for t in range(rows_w // 16):
                    dest2_v[pl.ds(t * 16, 16)] = (
                        dest_v[pl.ds(t * 16, 16)] + b * HW)
                pltpu.async_copy(rows_v, out_hbm.at[dest2_v], sem).wait()

    return _k(dest, lat_flat)


def _attn_body(n0_ref, x_ref, kv_ref, qflag_ref, o_ref,
               kn_ref, kvb_ref, *, bq, bk, c, nk, HW):
    i = pl.program_id(1)
    n0 = n0_ref[0, 0]

    @pl.when(i == 0)
    def _init_kv():
        kv = kv_ref[0]
        nrm = jnp.sqrt(jnp.sum(kv * kv, axis=1, keepdims=True)) + _EPS
        kn_ref[...] = (kv / nrm).astype(jnp.bfloat16)
        rows = lax.broadcasted_iota(jnp.int32, (HW, 1), 0)
        m = (rows < n0).astype(jnp.float32)
        kvb_ref[:, :c] = (kv * m).astype(jnp.bfloat16)
        kvb_ref[:, c:] = jnp.broadcast_to(m, (HW, 8)).astype(jnp.bfloat16)

    fm = x_ref[0, :c, :]
    qn = (fm / (jnp.sqrt(jnp.sum(fm * fm, axis=0, keepdims=True)) + _EPS)
          ).astype(jnp.bfloat16)

    def chunk(jj):
        kn_blk = kn_ref[pl.ds(jj * bk, bk), :]
        scores = lax.dot_general(
            qn, kn_blk, (((0,), (1,)), ((), ())),
            preferred_element_type=jnp.float32)
        p = jnp.exp(scores).astype(jnp.bfloat16)
        kvb_blk = kvb_ref[pl.ds(jj * bk, bk), :]
        return lax.dot_general(
            kvb_blk, p, (((0,), (1,)), ((), ())),
            preferred_element_type=jnp.float32)

    acc = chunk(0)
    for jj in range(1, nk):
        acc = lax.cond(
            jj * bk < n0,
            lambda acc=acc, jj=jj: acc + chunk(jj),
            lambda acc=acc: acc)

    scale = qflag_ref[0] / acc[c, :]
    o_ref[0, :2 * c, :] = x_ref[0]
    o_ref[0, 2 * c:, :] = acc[:c, :] * scale[None, :]


def _shift_concat(inp_chw, kvc, qflag, n0, *, bq, bk):
    B, C, HW = inp_chw.shape
    c = C // 2
    nk = HW // bk
    grid = (B, HW // bq)
    return pl.pallas_call(
        functools.partial(_attn_body, bq=bq, bk=bk, c=c, nk=nk, HW=HW),
        grid=grid,
        in_specs=[
            pl.BlockSpec(memory_space=pltpu.SMEM),
            pl.BlockSpec((1, C, bq), lambda b, i: (b, 0, i)),
            pl.BlockSpec((1, HW, c), lambda b, i: (b, 0, 0)),
            pl.BlockSpec((1, bq), lambda b, i: (0, i)),
        ],
        out_specs=pl.BlockSpec((1, 3 * c, bq), lambda b, i: (b, 0, i)),
        out_shape=jax.ShapeDtypeStruct((B, 3 * c, HW), jnp.float32),
        scratch_shapes=[
            pltpu.VMEM((HW, c), jnp.bfloat16),
            pltpu.VMEM((HW, c + 8), jnp.bfloat16),
        ],
        compiler_params=pltpu.CompilerParams(
            dimension_semantics=("arbitrary", "arbitrary"),
        ),
    )(n0, inp_chw, kvc, qflag)


def kernel(input, mask, shift_sz, stride, triple_w, flag):
    B, C, H, W = input.shape
    c = C // 2
    HW = H * W
    inp_chw = input.reshape(B, C, HW)
    qflag = flag.astype(jnp.float32).reshape(1, HW)

    lat_rm, dest, n0 = _latter_rowmajor(inp_chw, qflag)
    kvc = _sc_compact_scatter(dest.reshape(HW), lat_rm.reshape(B * HW, c),
                              B=B, HW=HW, c=c)
    out = _shift_concat(inp_chw, kvc.reshape(B, HW, c), qflag, n0,
                        bq=4096, bk=512)
    return out.reshape(B, C + c, H, W)


# speedup vs baseline: 1.0736x; 1.0736x over previous
import functools

import jax
import jax.numpy as jnp
from jax import lax
from jax.experimental import pallas as pl
from jax.experimental.pallas import tpu as pltpu
from jax.experimental.pallas import tpu_sc as plsc

_EPS = 1e-8
_NEG = -1e30


def _transpose_body(flag_ref, x_ref, o_ref, dest_ref, n0_ref, *, HW):
    b = pl.program_id(0)
    i = pl.program_id(1)

    @pl.when((b == 0) & (i == 0))
    def _ranks():
        z = jnp.where(flag_ref[...] < 0.5, 1.0, 0.0)
        cz = z
        sh = 1
        while sh < HW:
            shifted = jnp.concatenate(
                [jnp.zeros((1, sh), jnp.float32), cz[:, :HW - sh]], axis=1)
            cz = cz + shifted
            sh *= 2
        rank0 = cz - z
        total0 = jnp.sum(z)
        pos = lax.broadcasted_iota(jnp.int32, (1, HW), 1).astype(jnp.float32)
        dest = jnp.where(z > 0.5, rank0, total0 + (pos - rank0))
        dest_ref[...] = dest.astype(jnp.int32)
        n0_ref[...] = jnp.full((1, 8), total0, jnp.float32).astype(jnp.int32)

    o_ref[0] = jnp.swapaxes(x_ref[0], 0, 1)


def _latter_rowmajor(inp_chw, qflag, *, bqt=512):
    B, C, HW = inp_chw.shape
    c = C // 2
    return pl.pallas_call(
        functools.partial(_transpose_body, HW=HW),
        grid=(B, HW // bqt),
        in_specs=[
            pl.BlockSpec((1, HW), lambda b, i: (0, 0)),
            pl.BlockSpec((1, c, bqt), lambda b, i: (b, 1, i)),
        ],
        out_specs=[
            pl.BlockSpec((1, bqt, c), lambda b, i: (b, i, 0)),
            pl.BlockSpec((1, HW), lambda b, i: (0, 0)),
            pl.BlockSpec((1, 8), lambda b, i: (0, 0)),
        ],
        out_shape=[
            jax.ShapeDtypeStruct((B, HW, c), jnp.float32),
            jax.ShapeDtypeStruct((1, HW), jnp.int32),
            jax.ShapeDtypeStruct((1, 8), jnp.int32),
        ],
        compiler_params=pltpu.CompilerParams(
            dimension_semantics=("arbitrary", "arbitrary"),
        ),
    )(qflag, inp_chw)


def _sc_compact_scatter(dest, lat_flat, *, B, HW, c):
    info = plsc.get_sparse_core_info()
    NC, NS = info.num_cores, info.num_subcores
    NW = NC * NS
    rows_w = HW // NW

    mesh = plsc.VectorSubcoreMesh(core_axis_name="c", subcore_axis_name="s")

    @functools.partial(
        pl.kernel, mesh=mesh,
        out_type=jax.ShapeDtypeStruct((B * HW, c), jnp.float32),
        scratch_types=[
            pltpu.VMEM((rows_w,), jnp.int32),
            pltpu.VMEM((rows_w,), jnp.int32),
            pltpu.VMEM((rows_w, c), jnp.float32),
            pltpu.SemaphoreType.DMA,
        ],
    )
    def _k(dest_hbm, lat_hbm, out_hbm, dest_v, dest2_v, rows_v, sem):
        wid = lax.axis_index("s") * NC + lax.axis_index("c")
        base = pl.multiple_of(wid * rows_w, rows_w)
        pltpu.sync_copy(dest_hbm.at[pl.ds(base, rows_w)], dest_v)
        for b in range(B):
            pltpu.sync_copy(lat_hbm.at[pl.ds(b * HW + base, rows_w)], rows_v)
            if b == 0:
                pltpu.async_copy(rows_v, out_hbm.at[dest_v], sem).wait()
            else:
                for t in range(rows_w // 16):
                    dest2_v[pl.ds(t * 16, 16)] = (
                        dest_v[pl.ds(t * 16, 16)] + b * HW)
                pltpu.async_copy(rows_v, out_hbm.at[dest2_v], sem).wait()

    return _k(dest, lat_flat)


def _attn_body(n0_ref, x_ref, kv_ref, qflag_ref, o_ref,
               kn_ref, kvb_ref, *, bq, bk, c, nk, HW):
    i = pl.program_id(1)
    n0 = n0_ref[0, 0]

    @pl.when(i == 0)
    def _init_kv():
        kv = kv_ref[0]
        nrm = jnp.sqrt(jnp.sum(kv * kv, axis=1, keepdims=True)) + _EPS
        kn_ref[...] = (kv / nrm).astype(jnp.bfloat16)
        rows = lax.broadcasted_iota(jnp.int32, (HW, 1), 0)
        m = (rows < n0).astype(jnp.float32)
        kvb_ref[:, :c] = (kv * m).astype(jnp.bfloat16)
        kvb_ref[:, c:] = jnp.broadcast_to(m, (HW, 8)).astype(jnp.bfloat16)

    fm = x_ref[0, :c, :]
    qn = (fm / (jnp.sqrt(jnp.sum(fm * fm, axis=0, keepdims=True)) + _EPS)
          ).astype(jnp.bfloat16)

    def chunk(jj):
        kn_blk = kn_ref[pl.ds(jj * bk, bk), :]
        scores = lax.dot_general(
            qn, kn_blk, (((0,), (1,)), ((), ())),
            preferred_element_type=jnp.float32)
        p = jnp.exp(scores).astype(jnp.bfloat16)
        kvb_blk = kvb_ref[pl.ds(jj * bk, bk), :]
        return lax.dot_general(
            kvb_blk, p, (((0,), (1,)), ((), ())),
            preferred_element_type=jnp.float32)

    acc = chunk(0)
    for jj in range(1, nk):
        acc = lax.cond(
            jj * bk < n0,
            lambda acc=acc, jj=jj: acc + chunk(jj),
            lambda acc=acc: acc)

    scale = qflag_ref[0] / acc[c, :]
    o_ref[0, :2 * c, :] = x_ref[0]
    o_ref[0, 2 * c:, :] = acc[:c, :] * scale[None, :]


def _shift_concat(inp_chw, kvc, qflag, n0, *, bq, bk):
    B, C, HW = inp_chw.shape
    c = C // 2
    nk = HW // bk
    grid = (B, HW // bq)
    return pl.pallas_call(
        functools.partial(_attn_body, bq=bq, bk=bk, c=c, nk=nk, HW=HW),
        grid=grid,
        in_specs=[
            pl.BlockSpec(memory_space=pltpu.SMEM),
            pl.BlockSpec((1, C, bq), lambda b, i: (b, 0, i)),
            pl.BlockSpec((1, HW, c), lambda b, i: (b, 0, 0)),
            pl.BlockSpec((1, bq), lambda b, i: (0, i)),
        ],
        out_specs=pl.BlockSpec((1, 3 * c, bq), lambda b, i: (b, 0, i)),
        out_shape=jax.ShapeDtypeStruct((B, 3 * c, HW), jnp.float32),
        scratch_shapes=[
            pltpu.VMEM((HW, c), jnp.bfloat16),
            pltpu.VMEM((HW, c + 8), jnp.bfloat16),
        ],
        compiler_params=pltpu.CompilerParams(
            dimension_semantics=("arbitrary", "arbitrary"),
        ),
    )(n0, inp_chw, kvc, qflag)


def kernel(input, mask, shift_sz, stride, triple_w, flag):
    B, C, H, W = input.shape
    c = C // 2
    HW = H * W
    inp_chw = input.reshape(B, C, HW)
    qflag = flag.astype(jnp.float32).reshape(1, HW)

    lat_rm, dest, n0 = _latter_rowmajor(inp_chw, qflag)
    kvc = _sc_compact_scatter(dest.reshape(HW), lat_rm.reshape(B * HW, c),
                              B=B, HW=HW, c=c)
    out = _shift_concat(inp_chw, kvc.reshape(B, HW, c), qflag, n0,
                        bq=2048, bk=1024)
    return out.reshape(B, C + c, H, W)


# speedup vs baseline: 1.1126x; 1.0363x over previous
import functools

import jax
import jax.numpy as jnp
from jax import lax
from jax.experimental import pallas as pl
from jax.experimental.pallas import tpu as pltpu
from jax.experimental.pallas import tpu_sc as plsc

_EPS = 1e-8
_NEG = -1e30


def _transpose_body(flag_ref, x_ref, o_ref, dest_ref, n0_ref, *, HW):
    b = pl.program_id(0)
    i = pl.program_id(1)

    @pl.when((b == 0) & (i == 0))
    def _ranks():
        z = jnp.where(flag_ref[...] < 0.5, 1.0, 0.0)
        cz = z
        sh = 1
        while sh < HW:
            shifted = jnp.concatenate(
                [jnp.zeros((1, sh), jnp.float32), cz[:, :HW - sh]], axis=1)
            cz = cz + shifted
            sh *= 2
        rank0 = cz - z
        total0 = jnp.sum(z)
        pos = lax.broadcasted_iota(jnp.int32, (1, HW), 1).astype(jnp.float32)
        dest = jnp.where(z > 0.5, rank0, total0 + (pos - rank0))
        dest_ref[...] = dest.astype(jnp.int32)
        n0_ref[...] = jnp.full((1, 8), total0, jnp.float32).astype(jnp.int32)

    o_ref[0] = jnp.swapaxes(x_ref[0], 0, 1)


def _latter_rowmajor(inp_chw, qflag, *, bqt=512):
    B, C, HW = inp_chw.shape
    c = C // 2
    return pl.pallas_call(
        functools.partial(_transpose_body, HW=HW),
        grid=(B, HW // bqt),
        in_specs=[
            pl.BlockSpec((1, HW), lambda b, i: (0, 0)),
            pl.BlockSpec((1, c, bqt), lambda b, i: (b, 1, i)),
        ],
        out_specs=[
            pl.BlockSpec((1, bqt, c), lambda b, i: (b, i, 0)),
            pl.BlockSpec((1, HW), lambda b, i: (0, 0)),
            pl.BlockSpec((1, 8), lambda b, i: (0, 0)),
        ],
        out_shape=[
            jax.ShapeDtypeStruct((B, HW, c), jnp.float32),
            jax.ShapeDtypeStruct((1, HW), jnp.int32),
            jax.ShapeDtypeStruct((1, 8), jnp.int32),
        ],
        compiler_params=pltpu.CompilerParams(
            dimension_semantics=("arbitrary", "arbitrary"),
        ),
    )(qflag, inp_chw)


def _sc_compact_scatter(dest, lat_flat, *, B, HW, c):
    info = plsc.get_sparse_core_info()
    NC, NS = info.num_cores, info.num_subcores
    NW = NC * NS
    rows_w = HW // NW

    mesh = plsc.VectorSubcoreMesh(core_axis_name="c", subcore_axis_name="s")

    @functools.partial(
        pl.kernel, mesh=mesh,
        out_type=jax.ShapeDtypeStruct((B * HW, c), jnp.float32),
        scratch_types=[
            pltpu.VMEM((rows_w,), jnp.int32),
            pltpu.VMEM((rows_w,), jnp.int32),
            pltpu.VMEM((rows_w, c), jnp.float32),
            pltpu.SemaphoreType.DMA,
        ],
    )
    def _k(dest_hbm, lat_hbm, out_hbm, dest_v, dest2_v, rows_v, sem):
        wid = lax.axis_index("s") * NC + lax.axis_index("c")
        base = pl.multiple_of(wid * rows_w, rows_w)
        pltpu.sync_copy(dest_hbm.at[pl.ds(base, rows_w)], dest_v)
        for b in range(B):
            pltpu.sync_copy(lat_hbm.at[pl.ds(b * HW + base, rows_w)], rows_v)
            if b == 0:
                pltpu.async_copy(rows_v, out_hbm.at[dest_v], sem).wait()
            else:
                for t in range(rows_w // 16):
                    dest2_v[pl.ds(t * 16, 16)] = (
                        dest_v[pl.ds(t * 16, 16)] + b * HW)
                pltpu.async_copy(rows_v, out_hbm.at[dest2_v], sem).wait()

    return _k(dest, lat_flat)


def _attn_body(n0_ref, x_ref, kv_ref, qflag_ref, o_ref,
               kn_ref, kvb_ref, *, bq, bk, c, nk, HW):
    i = pl.program_id(1)
    n0 = n0_ref[0, 0]

    @pl.when(i == 0)
    def _init_kv():
        kv = kv_ref[0]
        nrm = jnp.sqrt(jnp.sum(kv * kv, axis=1, keepdims=True)) + _EPS
        kn_ref[...] = (kv / nrm).astype(jnp.bfloat16)
        rows = lax.broadcasted_iota(jnp.int32, (HW, 1), 0)
        m = (rows < n0).astype(jnp.float32)
        kvb_ref[:, :c] = (kv * m).astype(jnp.bfloat16)
        kvb_ref[:, c:] = jnp.broadcast_to(m, (HW, 8)).astype(jnp.bfloat16)

    fm = x_ref[0, :c, :]
    qn = (fm / (jnp.sqrt(jnp.sum(fm * fm, axis=0, keepdims=True)) + _EPS)
          ).astype(jnp.bfloat16)

    def chunk(jj):
        kn_blk = kn_ref[pl.ds(jj * bk, bk), :]
        scores = lax.dot_general(
            qn, kn_blk, (((0,), (1,)), ((), ())),
            preferred_element_type=jnp.float32)
        p = jnp.exp(scores).astype(jnp.bfloat16)
        kvb_blk = kvb_ref[pl.ds(jj * bk, bk), :]
        return lax.dot_general(
            kvb_blk, p, (((0,), (1,)), ((), ())),
            preferred_element_type=jnp.float32)

    acc = chunk(0)
    for jj in range(1, nk):
        acc = lax.cond(
            jj * bk < n0,
            lambda acc=acc, jj=jj: acc + chunk(jj),
            lambda acc=acc: acc)

    scale = qflag_ref[0] / acc[c, :]
    o_ref[0, :2 * c, :] = x_ref[0]
    o_ref[0, 2 * c:, :] = acc[:c, :] * scale[None, :]


def _shift_concat(inp_chw, kvc, qflag, n0, *, bq, bk):
    B, C, HW = inp_chw.shape
    c = C // 2
    nk = HW // bk
    grid = (B, HW // bq)
    return pl.pallas_call(
        functools.partial(_attn_body, bq=bq, bk=bk, c=c, nk=nk, HW=HW),
        grid=grid,
        in_specs=[
            pl.BlockSpec(memory_space=pltpu.SMEM),
            pl.BlockSpec((1, C, bq), lambda b, i: (b, 0, i)),
            pl.BlockSpec((1, HW, c), lambda b, i: (b, 0, 0)),
            pl.BlockSpec((1, bq), lambda b, i: (0, i)),
        ],
        out_specs=pl.BlockSpec((1, 3 * c, bq), lambda b, i: (b, 0, i)),
        out_shape=jax.ShapeDtypeStruct((B, 3 * c, HW), jnp.float32),
        scratch_shapes=[
            pltpu.VMEM((HW, c), jnp.bfloat16),
            pltpu.VMEM((HW, c + 8), jnp.bfloat16),
        ],
        compiler_params=pltpu.CompilerParams(
            dimension_semantics=("arbitrary", "arbitrary"),
        ),
    )(n0, inp_chw, kvc, qflag)


def kernel(input, mask, shift_sz, stride, triple_w, flag):
    B, C, H, W = input.shape
    c = C // 2
    HW = H * W
    inp_chw = input.reshape(B, C, HW)
    qflag = flag.astype(jnp.float32).reshape(1, HW)

    lat_rm, dest, n0 = _latter_rowmajor(inp_chw, qflag)
    kvc = _sc_compact_scatter(dest.reshape(HW), lat_rm.reshape(B * HW, c),
                              B=B, HW=HW, c=c)
    out = _shift_concat(inp_chw, kvc.reshape(B, HW, c), qflag, n0,
                        bq=2048, bk=2048)
    return out.reshape(B, C + c, H, W)
